# Initial kernel scaffold; baseline (speedup 1.0000x reference)
import jax
import jax.numpy as jnp
from jax.experimental import pallas as pl


def kernel(conv1_w, conv1_b, conv2_w, conv2_b, fc1_w, fc1_b, fc2_w, fc2_b, x):
    raise NotImplementedError("write your pallas kernel here")


# speedup vs baseline: 72.3576x; 72.3576x over previous
import jax
import jax.numpy as jnp
from jax import lax
from jax.experimental import pallas as pl
from jax.experimental.pallas import tpu as pltpu

BB = 1024
KS = 5
C1, C2 = 4, 8
OH1, OW1 = 24, 24
PH1, PW1 = 12, 12
OH2, OW2 = 8, 8
PH2, PW2 = 4, 4
NF = PH2 * PW2 * C2
F1 = 32
NC = 10


def _pool2x2_relu(v, ph, pw):
    h = v.reshape(ph, 2, pw, 2, 8, 128)
    m = jnp.maximum(jnp.maximum(h[:, 0, :, 0], h[:, 0, :, 1]),
                    jnp.maximum(h[:, 1, :, 0], h[:, 1, :, 1]))
    return jnp.maximum(m, 0.0)


def _fused_kernel(w1_ref, b1_ref, w2_ref, b2_ref, f1w_ref, f1b_ref,
                  f2w_ref, f2b_ref, x_ref, o_ref,
                  acc1_ref, p1_ref, acc2_ref, p2_ref):

    for c in range(C1):
        acc1_ref[c] = jnp.full((OH1, OW1, 8, 128), b1_ref[0, c], jnp.float32)

    def conv1_tap(r, carry):
        kh = r // KS
        kw = r % KS
        xw = x_ref[0, pl.ds(kh, OH1), pl.ds(kw, OW1)]
        for c in range(C1):
            acc1_ref[c] = acc1_ref[c] + xw * w1_ref[r, c]
        return carry

    lax.fori_loop(0, KS * KS, conv1_tap, 0)

    for c in range(C1):
        p1_ref[c] = _pool2x2_relu(acc1_ref[c], PH1, PW1)

    for co in range(C2):
        acc2_ref[co] = jnp.full((OH2, OW2, 8, 128), b2_ref[0, co], jnp.float32)

    def conv2_tap(r, carry):
        kh = r // KS
        kw = r % KS
        for ci in range(C1):
            win = p1_ref[ci, pl.ds(kh, OH2), pl.ds(kw, OW2)]
            for co in range(C2):
                acc2_ref[co] = acc2_ref[co] + win * w2_ref[r * C1 + ci, co]
        return carry

    lax.fori_loop(0, KS * KS, conv2_tap, 0)

    for co in range(C2):
        pc = _pool2x2_relu(acc2_ref[co], PH2, PW2)
        for hh in range(PH2):
            for ww in range(PW2):
                p2_ref[(hh * PW2 + ww) * C2 + co] = pc[hh, ww]

    p2 = p2_ref[...].reshape(NF, BB)
    y1 = jnp.dot(f1w_ref[...], p2,
                 preferred_element_type=jnp.float32) + f1b_ref[...]
    y1 = jnp.maximum(y1, 0.0)
    logits = jnp.dot(f2w_ref[...], y1,
                     preferred_element_type=jnp.float32) + f2b_ref[...]
    m = jnp.max(logits, axis=0, keepdims=True)
    z = logits - m
    lse = jnp.log(jnp.sum(jnp.exp(z), axis=0, keepdims=True))
    o_ref[0] = z - lse


def kernel(conv1_w, conv1_b, conv2_w, conv2_b, fc1_w, fc1_b, fc2_w, fc2_b, x):
    n = x.shape[0]
    pad = (-n) % BB
    x = x.astype(jnp.float32).reshape(n, 28, 28)
    if pad:
        x = jnp.concatenate([x, jnp.zeros((pad, 28, 28), jnp.float32)], axis=0)
    nblk = x.shape[0] // BB

    x_t = x.reshape(nblk, 8, 128, 28, 28).transpose(0, 3, 4, 1, 2)

    w1s = conv1_w.reshape(KS * KS, C1)
    w2s = conv2_w.reshape(KS * KS * C1, C2)
    f1wT = fc1_w.reshape(NF, F1).T
    f1bT = fc1_b.reshape(1, F1).T
    f2wT = fc2_w.T
    f2bT = fc2_b.reshape(1, NC).T

    out = pl.pallas_call(
        _fused_kernel,
        out_shape=jax.ShapeDtypeStruct((nblk, NC, BB), jnp.float32),
        grid=(nblk,),
        in_specs=[
            pl.BlockSpec(memory_space=pltpu.SMEM),
            pl.BlockSpec(memory_space=pltpu.SMEM),
            pl.BlockSpec(memory_space=pltpu.SMEM),
            pl.BlockSpec(memory_space=pltpu.SMEM),
            pl.BlockSpec((F1, NF), lambda i: (0, 0)),
            pl.BlockSpec((F1, 1), lambda i: (0, 0)),
            pl.BlockSpec((NC, F1), lambda i: (0, 0)),
            pl.BlockSpec((NC, 1), lambda i: (0, 0)),
            pl.BlockSpec((1, 28, 28, 8, 128), lambda i: (i, 0, 0, 0, 0)),
        ],
        out_specs=pl.BlockSpec((1, NC, BB), lambda i: (i, 0, 0)),
        scratch_shapes=[
            pltpu.VMEM((C1, OH1, OW1, 8, 128), jnp.float32),
            pltpu.VMEM((C1, PH1, PW1, 8, 128), jnp.float32),
            pltpu.VMEM((C2, OH2, OW2, 8, 128), jnp.float32),
            pltpu.VMEM((NF, 8, 128), jnp.float32),
        ],
        compiler_params=pltpu.CompilerParams(
            dimension_semantics=("parallel",),
            vmem_limit_bytes=48 * 1024 * 1024),
    )(w1s, conv1_b, w2s, conv2_b, f1wT, f1bT, f2wT, f2bT, x_t)

    return out.transpose(0, 2, 1).reshape(nblk * BB, NC)[:n]


# speedup vs baseline: 116.0109x; 1.6033x over previous
import jax
import jax.numpy as jnp
from jax import lax
from jax.experimental import pallas as pl
from jax.experimental.pallas import tpu as pltpu

BB = 1024
KS = 5
C1, C2 = 4, 8
OH1, OW1 = 24, 24
PH1, PW1 = 12, 12
OH2, OW2 = 8, 8
PH2, PW2 = 4, 4
NF = PH2 * PW2 * C2
F1 = 32
NC = 10


def _pool2x2_relu(v, ph, pw):
    h = v.reshape(ph, 2, pw, 2, 8, 128)
    m = jnp.maximum(jnp.maximum(h[:, 0, :, 0], h[:, 0, :, 1]),
                    jnp.maximum(h[:, 1, :, 0], h[:, 1, :, 1]))
    return jnp.maximum(m, 0.0)


def _fused_kernel(w1_ref, b1_ref, w2_ref, b2_ref, f1w_ref, f1b_ref,
                  f2w_ref, f2b_ref, x_ref, o_ref,
                  acc1_ref, p1_ref, acc2_ref, p2_ref):

    for c in range(C1):
        acc1_ref[c] = jnp.full((OH1, OW1, 8, 128), b1_ref[0, c], jnp.float32)

    def conv1_row(kh, carry):
        accs = [acc1_ref[c] for c in range(C1)]
        for kw in range(KS):
            xw = x_ref[0, pl.ds(kh, OH1), pl.ds(kw, OW1)]
            for c in range(C1):
                accs[c] = accs[c] + xw * w1_ref[kh * KS + kw, c]
        for c in range(C1):
            acc1_ref[c] = accs[c]
        return carry

    lax.fori_loop(0, KS, conv1_row, 0)

    for c in range(C1):
        p1_ref[c] = _pool2x2_relu(acc1_ref[c], PH1, PW1)

    for co in range(C2):
        acc2_ref[co] = jnp.full((OH2, OW2, 8, 128), b2_ref[0, co], jnp.float32)

    def conv2_row(kh, carry):
        accs = [acc2_ref[co] for co in range(C2)]
        for kw in range(KS):
            for ci in range(C1):
                win = p1_ref[ci, pl.ds(kh, OH2), pl.ds(kw, OW2)]
                for co in range(C2):
                    accs[co] = accs[co] + win * w2_ref[(kh * KS + kw) * C1 + ci, co]
        for co in range(C2):
            acc2_ref[co] = accs[co]
        return carry

    lax.fori_loop(0, KS, conv2_row, 0)

    for co in range(C2):
        pc = _pool2x2_relu(acc2_ref[co], PH2, PW2)
        for hh in range(PH2):
            for ww in range(PW2):
                p2_ref[(hh * PW2 + ww) * C2 + co] = pc[hh, ww]

    p2 = p2_ref[...].reshape(NF, BB)
    y1 = jnp.dot(f1w_ref[...], p2,
                 preferred_element_type=jnp.float32) + f1b_ref[...]
    y1 = jnp.maximum(y1, 0.0)
    logits = jnp.dot(f2w_ref[...], y1,
                     preferred_element_type=jnp.float32) + f2b_ref[...]
    m = jnp.max(logits, axis=0, keepdims=True)
    z = logits - m
    lse = jnp.log(jnp.sum(jnp.exp(z), axis=0, keepdims=True))
    o_ref[0] = z - lse


def kernel(conv1_w, conv1_b, conv2_w, conv2_b, fc1_w, fc1_b, fc2_w, fc2_b, x):
    n = x.shape[0]
    pad = (-n) % BB
    x = x.astype(jnp.float32).reshape(n, 28, 28)
    if pad:
        x = jnp.concatenate([x, jnp.zeros((pad, 28, 28), jnp.float32)], axis=0)
    nblk = x.shape[0] // BB

    x_t = x.reshape(nblk, 8, 128, 28, 28).transpose(0, 3, 4, 1, 2)

    w1s = conv1_w.reshape(KS * KS, C1)
    w2s = conv2_w.reshape(KS * KS * C1, C2)
    f1wT = fc1_w.reshape(NF, F1).T
    f1bT = fc1_b.reshape(1, F1).T
    f2wT = fc2_w.T
    f2bT = fc2_b.reshape(1, NC).T

    out = pl.pallas_call(
        _fused_kernel,
        out_shape=jax.ShapeDtypeStruct((nblk, NC, BB), jnp.float32),
        grid=(nblk,),
        in_specs=[
            pl.BlockSpec(memory_space=pltpu.SMEM),
            pl.BlockSpec(memory_space=pltpu.SMEM),
            pl.BlockSpec(memory_space=pltpu.SMEM),
            pl.BlockSpec(memory_space=pltpu.SMEM),
            pl.BlockSpec((F1, NF), lambda i: (0, 0)),
            pl.BlockSpec((F1, 1), lambda i: (0, 0)),
            pl.BlockSpec((NC, F1), lambda i: (0, 0)),
            pl.BlockSpec((NC, 1), lambda i: (0, 0)),
            pl.BlockSpec((1, 28, 28, 8, 128), lambda i: (i, 0, 0, 0, 0)),
        ],
        out_specs=pl.BlockSpec((1, NC, BB), lambda i: (i, 0, 0)),
        scratch_shapes=[
            pltpu.VMEM((C1, OH1, OW1, 8, 128), jnp.float32),
            pltpu.VMEM((C1, PH1, PW1, 8, 128), jnp.float32),
            pltpu.VMEM((C2, OH2, OW2, 8, 128), jnp.float32),
            pltpu.VMEM((NF, 8, 128), jnp.float32),
        ],
        compiler_params=pltpu.CompilerParams(
            dimension_semantics=("parallel",),
            vmem_limit_bytes=48 * 1024 * 1024),
    )(w1s, conv1_b, w2s, conv2_b, f1wT, f1bT, f2wT, f2bT, x_t)

    return out.transpose(0, 2, 1).reshape(nblk * BB, NC)[:n]
